# Initial kernel scaffold; baseline (speedup 1.0000x reference)
import jax
import jax.numpy as jnp
from jax.experimental import pallas as pl


def kernel(inputs, mask):
    raise NotImplementedError("write your pallas kernel here")


# speedup vs baseline: 1.3301x; 1.3301x over previous
import functools

import jax
import jax.numpy as jnp
from jax.experimental import pallas as pl
from jax.experimental.pallas import tpu as pltpu

N_GROUPS = 100
TOTAL = 100000
BATCH = 1024
ROWS_PER_STEP = 8


def _copy_body(idx_ref, mask_ref, out_ref):
    i = pl.program_id(0)
    for k in range(ROWS_PER_STEP):
        g = idx_ref[i * ROWS_PER_STEP + k]
        out_ref[k, :] = mask_ref[g, :]


def kernel(inputs, mask):
    grid = (BATCH // ROWS_PER_STEP,)
    grid_spec = pltpu.PrefetchScalarGridSpec(
        num_scalar_prefetch=1,
        grid=grid,
        in_specs=[
            pl.BlockSpec((N_GROUPS, TOTAL), lambda i, idx_ref: (0, 0)),
        ],
        out_specs=pl.BlockSpec((ROWS_PER_STEP, TOTAL), lambda i, idx_ref: (i, 0)),
    )
    return pl.pallas_call(
        _copy_body,
        grid_spec=grid_spec,
        out_shape=jax.ShapeDtypeStruct((BATCH, TOTAL), jnp.bool_),
    )(inputs, mask)
